# baseline (speedup 1.0000x reference)
import functools

import jax
import jax.numpy as jnp
from jax import lax
from jax.experimental import pallas as pl
from jax.experimental.pallas import tpu as pltpu
from jax.experimental.pallas import tpu_sc as plsc

N = 10000
E = 320000
D_IN = 128
D_HID = 256
D_OUT = 128
DH = 128

NC = 2
NS = 16
CK = 40
BQ = 20
EC = 40
NH = 5120
RPH = NH // NS
AGG_CHUNKS = E // (NS * CK)

_mesh = plsc.VectorSubcoreMesh(core_axis_name="c", subcore_axis_name="s")
_f32 = jnp.float32


def _heron_dinv(cnt):
    x = cnt + 1.0
    s = 0.5 * (1.0 + x)
    for _ in range(14):
        s = 0.5 * (s + x / s)
    return s / x


def _fill_rows(buf, nrows, ncols, value):
    vv = jnp.full((16,), value, _f32)

    def body(r, _):
        for j in range(ncols // 16):
            buf[r, pl.ds(j * 16, 16)] = vv
        return 0

    lax.fori_loop(0, nrows, body, 0, unroll=False)


def _zero_acc_slice(zbuf, acc_sc, s):
    def body(i, _):
        pltpu.sync_copy(zbuf, acc_sc.at[pl.ds(s * RPH + i * CK, CK)])
        return 0

    lax.fori_loop(0, RPH // CK, body, 0, unroll=False)


_HOUT = jax.ShapeDtypeStruct((N, DH), _f32)
_DOUT = jax.ShapeDtypeStruct((N, 16), _f32)


@functools.partial(
    pl.kernel,
    out_type=(_HOUT,) * 6 + (_DOUT,) * 2,
    mesh=_mesh,
    scratch_types=[
        pltpu.VMEM((CK, DH), _f32),
        pltpu.VMEM((CK, DH), _f32),
        pltpu.VMEM((EC, DH), _f32),
        pltpu.VMEM((EC, 16), _f32),
        pltpu.VMEM((16, DH), _f32),
        pltpu.VMEM((BQ, 1, CK), jnp.int32),
        pltpu.VMEM((BQ, 1, CK), jnp.int32),
        pltpu.VMEM_SHARED((NH, DH), _f32),
        pltpu.SemaphoreType.DMA,
        pltpu.SemaphoreType.DMA,
        pltpu.SemaphoreType.DMA,
        pltpu.SemaphoreType.DMA,
    ],
)
def _mega_kernel(srcm0, srcm1, dstm0, dstm1, xw, b1rlo, b1rhi,
                 y1lo, y1hi, y2lo, y2hi, glo, ghi, dvlo, dvhi,
                 rows, rowsb, yv, dvb, b1v, sbufB, dbufB,
                 acc_sc, semg0, semg1, sems, sems2):
    c = lax.axis_index("c")
    s = lax.axis_index("s")
    srcms = (srcm0, srcm1)
    dstms = (dstm0, dstm1)
    bufs = (rows, rowsb)
    semgs = (semg0, semg1)

    def nchunks(h):
        if h == 0:
            return RPH // EC
        return jnp.where(s == NS - 1, (N - NH - (NS - 1) * RPH) // EC,
                         RPH // EC)

    def deg_scatter(h):
        _fill_rows(rows, CK, DH, 0.0)
        _zero_acc_slice(rows, acc_sc, s)
        _fill_rows(rows, CK, DH, 1.0)
        plsc.subcore_barrier()

        def batch(jb, _):
            pltpu.sync_copy(dstms[h].at[s, pl.ds(jb * BQ, BQ)], dbufB)

            def body(g, _):
                cp0 = pltpu.async_copy(
                    rows,
                    acc_sc.at[plsc.Indices(dbufB.at[2 * g, 0],
                                           ignored_value=-1)],
                    sems, add=True)
                cp1 = pltpu.async_copy(
                    rows,
                    acc_sc.at[plsc.Indices(dbufB.at[2 * g + 1, 0],
                                           ignored_value=-1)],
                    sems2, add=True)
                cp0.wait()
                cp1.wait()
                return 0

            lax.fori_loop(0, BQ // 2, body, 0, unroll=False)
            return 0

        lax.fori_loop(0, AGG_CHUNKS // BQ, batch, 0, unroll=False)
        plsc.subcore_barrier()

    def deg_elementwise(h, xoff, y1t, dvt):
        g0 = h * NH + s * RPH

        def chunk(k, _):
            pltpu.sync_copy(acc_sc.at[pl.ds(s * RPH + k * EC, EC)],
                            rows.at[pl.ds(0, EC)])
            pltpu.sync_copy(xw.at[pl.ds(g0 + k * EC, EC), pl.ds(xoff, DH)],
                            yv)

            def row(r, _):
                dv = _heron_dinv(rows[r, pl.ds(0, 16)])
                dvb[r, :] = dv
                for j in range(DH // 16):
                    sl = pl.ds(j * 16, 16)
                    yv[r, sl] = dv * yv[r, sl]
                return 0

            lax.fori_loop(0, EC, row, 0, unroll=False)
            pltpu.sync_copy(yv, y1t.at[pl.ds(g0 + k * EC, EC)])
            pltpu.sync_copy(dvb, dvt.at[pl.ds(g0 + k * EC, EC)])
            return 0

        lax.fori_loop(0, nchunks(h), chunk, 0, unroll=False)

    def agg_scatter(h, tbl):
        _fill_rows(rows, CK, DH, 0.0)
        _zero_acc_slice(rows, acc_sc, s)
        plsc.subcore_barrier()

        def batch(jb, _):
            pltpu.sync_copy(srcms[h].at[s, pl.ds(jb * BQ, BQ)], sbufB)
            pltpu.sync_copy(dstms[h].at[s, pl.ds(jb * BQ, BQ)], dbufB)
            for b in (0, 1):
                pltpu.async_copy(
                    tbl.at[plsc.Indices(sbufB.at[b, 0], ignored_value=-1)],
                    bufs[b], semgs[b])

            def body(g, _):
                q0 = 2 * g
                q1 = q0 + 1
                pltpu.make_async_copy(
                    tbl.at[plsc.Indices(sbufB.at[q0, 0], ignored_value=-1)],
                    bufs[0], semgs[0]).wait()
                cp0 = pltpu.async_copy(
                    bufs[0],
                    acc_sc.at[plsc.Indices(dbufB.at[q0, 0],
                                           ignored_value=-1)],
                    sems, add=True)
                pltpu.make_async_copy(
                    tbl.at[plsc.Indices(sbufB.at[q1, 0], ignored_value=-1)],
                    bufs[1], semgs[1]).wait()
                cp1 = pltpu.async_copy(
                    bufs[1],
                    acc_sc.at[plsc.Indices(dbufB.at[q1, 0],
                                           ignored_value=-1)],
                    sems2, add=True)
                cp0.wait()

                @pl.when(q0 + 2 < BQ)
                def _():
                    pltpu.async_copy(
                        tbl.at[plsc.Indices(sbufB.at[q0 + 2, 0],
                                            ignored_value=-1)],
                        bufs[0], semgs[0])

                cp1.wait()

                @pl.when(q1 + 2 < BQ)
                def _():
                    pltpu.async_copy(
                        tbl.at[plsc.Indices(sbufB.at[q1 + 2, 0],
                                            ignored_value=-1)],
                        bufs[1], semgs[1])
                return 0

            lax.fori_loop(0, BQ // 2, body, 0, unroll=False)
            return 0

        lax.fori_loop(0, AGG_CHUNKS // BQ, batch, 0, unroll=False)
        plsc.subcore_barrier()

    def layer1_elementwise(h, y1t, y2t, dvt):
        g0 = h * NH + s * RPH

        def chunk(k, _):
            pltpu.sync_copy(acc_sc.at[pl.ds(s * RPH + k * EC, EC)],
                            rows.at[pl.ds(0, EC)])
            pltpu.sync_copy(y1t.at[pl.ds(g0 + k * EC, EC)], yv)
            pltpu.sync_copy(dvt.at[pl.ds(g0 + k * EC, EC)], dvb)

            def row(r, _):
                dv = dvb[r, :]
                for j in range(DH // 16):
                    sl = pl.ds(j * 16, 16)
                    hv = jnp.maximum(
                        dv * (rows[r, sl] + yv[r, sl]) + b1v[0, sl], 0.0)
                    yv[r, sl] = dv * hv
                return 0

            lax.fori_loop(0, EC, row, 0, unroll=False)
            pltpu.sync_copy(yv, y2t.at[pl.ds(g0 + k * EC, EC)])
            return 0

        lax.fori_loop(0, nchunks(h), chunk, 0, unroll=False)

    def g_elementwise(h, y2t, gt, dvt):
        g0 = h * NH + s * RPH

        def chunk(k, _):
            pltpu.sync_copy(acc_sc.at[pl.ds(s * RPH + k * EC, EC)],
                            rows.at[pl.ds(0, EC)])
            pltpu.sync_copy(y2t.at[pl.ds(g0 + k * EC, EC)], yv)
            pltpu.sync_copy(dvt.at[pl.ds(g0 + k * EC, EC)], dvb)

            def row(r, _):
                dv = dvb[r, :]
                for j in range(DH // 16):
                    sl = pl.ds(j * 16, 16)
                    yv[r, sl] = dv * (rows[r, sl] + yv[r, sl])
                return 0

            lax.fori_loop(0, EC, row, 0, unroll=False)
            pltpu.sync_copy(yv, gt.at[pl.ds(g0 + k * EC, EC)])
            return 0

        lax.fori_loop(0, nchunks(h), chunk, 0, unroll=False)

    def run(xoff, y1t, y2t, gt, dvt, b1r):
        pltpu.sync_copy(b1r, b1v)
        for h in (0, 1):
            deg_scatter(h)
            deg_elementwise(h, xoff, y1t, dvt)
        for h in (0, 1):
            agg_scatter(h, y1t)
            layer1_elementwise(h, y1t, y2t, dvt)
        for h in (0, 1):
            agg_scatter(h, y2t)
            g_elementwise(h, y2t, gt, dvt)

    @pl.when(c == 0)
    def _():
        run(0, y1lo, y2lo, glo, dvlo, b1rlo)

    @pl.when(c == 1)
    def _():
        run(DH, y1hi, y2hi, ghi, dvhi, b1rhi)


BN = 1000


def _mm_body(x_ref, w_ref, o_ref):
    o_ref[...] = jnp.dot(x_ref[...], w_ref[...],
                         preferred_element_type=_f32)


def _tc_matmul(x, w):
    m, k = x.shape
    n = w.shape[1]
    return pl.pallas_call(
        _mm_body,
        grid=(m // BN,),
        in_specs=[pl.BlockSpec((BN, k), lambda i: (i, 0)),
                  pl.BlockSpec((k, n), lambda i: (0, 0))],
        out_specs=pl.BlockSpec((BN, n), lambda i: (i, 0)),
        out_shape=jax.ShapeDtypeStruct((m, n), _f32),
    )(x, w)


def _mask_body(src_ref, dst_ref, s0_ref, s1_ref, d0_ref, d1_ref):
    src = src_ref[...]
    dst = dst_ref[...]
    neg1 = jnp.full(src.shape, -1, jnp.int32)
    v0 = dst < NH
    s0_ref[...] = jnp.where(v0, src, neg1)
    d0_ref[...] = jnp.where(v0, dst, neg1)
    s1_ref[...] = jnp.where(v0, neg1, src)
    d1_ref[...] = jnp.where(v0, neg1, dst - NH)


def _tc_mask(src2, dst2):
    nrows = src2.shape[0]
    bspec = pl.BlockSpec((nrows // 10, CK), lambda i: (i, 0))
    return pl.pallas_call(
        _mask_body,
        grid=(10,),
        in_specs=[bspec, bspec],
        out_specs=[bspec] * 4,
        out_shape=[jax.ShapeDtypeStruct((nrows, CK), jnp.int32)] * 4,
    )(src2, dst2)


def _final_body(glo_ref, ghi_ref, wmu_a_ref, wmu_b_ref, bmu_ref,
                wls_a_ref, wls_b_ref, bls_ref, mu_ref, ls_ref):
    glo = glo_ref[...]
    ghi = ghi_ref[...]
    mu_ref[...] = (jnp.dot(glo, wmu_a_ref[...], preferred_element_type=_f32)
                   + jnp.dot(ghi, wmu_b_ref[...], preferred_element_type=_f32)
                   + bmu_ref[...])
    ls_ref[...] = (jnp.dot(glo, wls_a_ref[...], preferred_element_type=_f32)
                   + jnp.dot(ghi, wls_b_ref[...], preferred_element_type=_f32)
                   + bls_ref[...])


def _tc_final(glo, ghi, wmu_a, wmu_b, bmu, wls_a, wls_b, bls):
    hspec = pl.BlockSpec((BN, DH), lambda i: (i, 0))
    wspec = pl.BlockSpec((DH, D_OUT), lambda i: (0, 0))
    bspec = pl.BlockSpec((1, D_OUT), lambda i: (0, 0))
    return pl.pallas_call(
        _final_body,
        grid=(N // BN,),
        in_specs=[hspec, hspec, wspec, wspec, bspec, wspec, wspec, bspec],
        out_specs=[pl.BlockSpec((BN, D_OUT), lambda i: (i, 0))] * 2,
        out_shape=[jax.ShapeDtypeStruct((N, D_OUT), _f32)] * 2,
    )(glo, ghi, wmu_a, wmu_b, bmu, wls_a, wls_b, bls)


def kernel(x, edge_index, W1, b1, Wmu, bmu, Wls, bls):
    src2 = edge_index[0].reshape(E // CK, CK)
    dst2 = edge_index[1].reshape(E // CK, CK)

    sm0, sm1, dm0, dm1 = _tc_mask(src2, dst2)
    idx4 = lambda a: a.reshape(NS, AGG_CHUNKS, 1, CK)

    xw = _tc_matmul(x, W1)

    b1rlo = jnp.broadcast_to(b1[:DH], (16, DH))
    b1rhi = jnp.broadcast_to(b1[DH:], (16, DH))
    outs = _mega_kernel(idx4(sm0), idx4(sm1), idx4(dm0), idx4(dm1), xw,
                        b1rlo, b1rhi)
    glo, ghi = outs[4], outs[5]

    mu, ls = _tc_final(glo, ghi,
                       Wmu[:DH], Wmu[DH:], bmu.reshape(1, D_OUT),
                       Wls[:DH], Wls[DH:], bls.reshape(1, D_OUT))
    return (mu, ls)


# speedup vs baseline: 1.1057x; 1.1057x over previous
import functools

import jax
import jax.numpy as jnp
from jax import lax
from jax.experimental import pallas as pl
from jax.experimental.pallas import tpu as pltpu
from jax.experimental.pallas import tpu_sc as plsc

N = 10000
E = 320000
D_IN = 128
D_HID = 256
D_OUT = 128
DH = 128

NC = 2
NS = 16
CK = 40
BQ = 20
EC = 40
NH = 5120
RPH = NH // NS
AGG_CHUNKS = E // (NS * CK)

_mesh = plsc.VectorSubcoreMesh(core_axis_name="c", subcore_axis_name="s")
_f32 = jnp.float32


def _heron_dinv(cnt):
    x = cnt + 1.0
    s = 0.5 * (1.0 + x)
    for _ in range(14):
        s = 0.5 * (s + x / s)
    return s / x


def _fill_rows(buf, nrows, ncols, value):
    vv = jnp.full((16,), value, _f32)

    def body(r, _):
        for j in range(ncols // 16):
            buf[r, pl.ds(j * 16, 16)] = vv
        return 0

    lax.fori_loop(0, nrows, body, 0, unroll=False)


def _zero_acc_slice(zbuf, acc_sc, s):
    def body(i, _):
        pltpu.sync_copy(zbuf, acc_sc.at[pl.ds(s * RPH + i * CK, CK)])
        return 0

    lax.fori_loop(0, RPH // CK, body, 0, unroll=False)


_HOUT = jax.ShapeDtypeStruct((N, DH), _f32)
_DOUT = jax.ShapeDtypeStruct((N, 16), _f32)


@functools.partial(
    pl.kernel,
    out_type=(_HOUT,) * 6 + (_DOUT,) * 2,
    mesh=_mesh,
    scratch_types=[
        pltpu.VMEM((CK, DH), _f32),
        pltpu.VMEM((CK, DH), _f32),
        pltpu.VMEM((EC, DH), _f32),
        pltpu.VMEM((EC, 16), _f32),
        pltpu.VMEM((16, DH), _f32),
        pltpu.VMEM((BQ, 1, CK), jnp.int32),
        pltpu.VMEM((BQ, 1, CK), jnp.int32),
        pltpu.VMEM_SHARED((NH, DH), _f32),
        pltpu.VMEM_SHARED((NH, 16), _f32),
        pltpu.SemaphoreType.DMA,
        pltpu.SemaphoreType.DMA,
        pltpu.SemaphoreType.DMA,
        pltpu.SemaphoreType.DMA,
    ],
)
def _mega_kernel(srcm0, srcm1, dstm0, dstm1, xw, b1rlo, b1rhi,
                 y1lo, y1hi, y2lo, y2hi, glo, ghi, dvlo, dvhi,
                 rows, rowsb, yv, dvb, b1v, sbufB, dbufB,
                 acc_sc, deg_sc, semg0, semg1, sems, sems2):
    c = lax.axis_index("c")
    s = lax.axis_index("s")
    srcms = (srcm0, srcm1)
    dstms = (dstm0, dstm1)
    bufs = (rows, rowsb)
    semgs = (semg0, semg1)

    def nchunks(h):
        if h == 0:
            return RPH // EC
        return jnp.where(s == NS - 1, (N - NH - (NS - 1) * RPH) // EC,
                         RPH // EC)

    def deg_scatter(h):
        _fill_rows(dvb, EC, 16, 0.0)

        def zb(i, _):
            pltpu.sync_copy(dvb, deg_sc.at[pl.ds(s * RPH + i * EC, EC)])
            return 0

        lax.fori_loop(0, RPH // EC, zb, 0, unroll=False)
        _fill_rows(dvb, EC, 16, 1.0)
        plsc.subcore_barrier()

        def batch(jb, _):
            pltpu.sync_copy(dstms[h].at[s, pl.ds(jb * BQ, BQ)], dbufB)

            def body(g, _):
                cp0 = pltpu.async_copy(
                    dvb,
                    deg_sc.at[plsc.Indices(dbufB.at[2 * g, 0],
                                           ignored_value=-1)],
                    sems, add=True)
                cp1 = pltpu.async_copy(
                    dvb,
                    deg_sc.at[plsc.Indices(dbufB.at[2 * g + 1, 0],
                                           ignored_value=-1)],
                    sems2, add=True)
                cp0.wait()
                cp1.wait()
                return 0

            lax.fori_loop(0, BQ // 2, body, 0, unroll=False)
            return 0

        lax.fori_loop(0, AGG_CHUNKS // BQ, batch, 0, unroll=False)
        plsc.subcore_barrier()

    def deg_elementwise(h, xoff, y1t, dvt):
        g0 = h * NH + s * RPH

        def chunk(k, _):
            pltpu.sync_copy(deg_sc.at[pl.ds(s * RPH + k * EC, EC)], dvb)
            pltpu.sync_copy(xw.at[pl.ds(g0 + k * EC, EC), pl.ds(xoff, DH)],
                            yv)

            def row(r, _):
                dv = _heron_dinv(dvb[r, :])
                dvb[r, :] = dv
                for j in range(DH // 16):
                    sl = pl.ds(j * 16, 16)
                    yv[r, sl] = dv * yv[r, sl]
                return 0

            lax.fori_loop(0, EC, row, 0, unroll=False)
            pltpu.sync_copy(yv, y1t.at[pl.ds(g0 + k * EC, EC)])
            pltpu.sync_copy(dvb, dvt.at[pl.ds(g0 + k * EC, EC)])
            return 0

        lax.fori_loop(0, nchunks(h), chunk, 0, unroll=False)

    def agg_scatter(h, tbl):
        _fill_rows(rows, CK, DH, 0.0)
        _zero_acc_slice(rows, acc_sc, s)
        plsc.subcore_barrier()

        def batch(jb, _):
            pltpu.sync_copy(srcms[h].at[s, pl.ds(jb * BQ, BQ)], sbufB)
            pltpu.sync_copy(dstms[h].at[s, pl.ds(jb * BQ, BQ)], dbufB)
            for b in (0, 1):
                pltpu.async_copy(
                    tbl.at[plsc.Indices(sbufB.at[b, 0], ignored_value=-1)],
                    bufs[b], semgs[b])

            def body(g, _):
                for b in (0, 1):
                    q = 2 * g + b
                    pltpu.make_async_copy(
                        tbl.at[plsc.Indices(sbufB.at[q, 0],
                                            ignored_value=-1)],
                        bufs[b], semgs[b]).wait()
                    pltpu.async_copy(
                        bufs[b],
                        acc_sc.at[plsc.Indices(dbufB.at[q, 0],
                                               ignored_value=-1)],
                        sems, add=True).wait()

                    @pl.when(q + 2 < BQ)
                    def _(b=b, q=q):
                        pltpu.async_copy(
                            tbl.at[plsc.Indices(sbufB.at[q + 2, 0],
                                                ignored_value=-1)],
                            bufs[b], semgs[b])
                return 0

            lax.fori_loop(0, BQ // 2, body, 0, unroll=False)
            return 0

        lax.fori_loop(0, AGG_CHUNKS // BQ, batch, 0, unroll=False)
        plsc.subcore_barrier()

    def layer1_elementwise(h, y1t, y2t, dvt):
        g0 = h * NH + s * RPH

        def chunk(k, _):
            pltpu.sync_copy(acc_sc.at[pl.ds(s * RPH + k * EC, EC)],
                            rows.at[pl.ds(0, EC)])
            pltpu.sync_copy(y1t.at[pl.ds(g0 + k * EC, EC)], yv)
            pltpu.sync_copy(dvt.at[pl.ds(g0 + k * EC, EC)], dvb)

            def row(r, _):
                dv = dvb[r, :]
                for j in range(DH // 16):
                    sl = pl.ds(j * 16, 16)
                    hv = jnp.maximum(
                        dv * (rows[r, sl] + yv[r, sl]) + b1v[0, sl], 0.0)
                    yv[r, sl] = dv * hv
                return 0

            lax.fori_loop(0, EC, row, 0, unroll=False)
            pltpu.sync_copy(yv, y2t.at[pl.ds(g0 + k * EC, EC)])
            return 0

        lax.fori_loop(0, nchunks(h), chunk, 0, unroll=False)

    def g_elementwise(h, y2t, gt, dvt):
        g0 = h * NH + s * RPH

        def chunk(k, _):
            pltpu.sync_copy(acc_sc.at[pl.ds(s * RPH + k * EC, EC)],
                            rows.at[pl.ds(0, EC)])
            pltpu.sync_copy(y2t.at[pl.ds(g0 + k * EC, EC)], yv)
            pltpu.sync_copy(dvt.at[pl.ds(g0 + k * EC, EC)], dvb)

            def row(r, _):
                dv = dvb[r, :]
                for j in range(DH // 16):
                    sl = pl.ds(j * 16, 16)
                    yv[r, sl] = dv * (rows[r, sl] + yv[r, sl])
                return 0

            lax.fori_loop(0, EC, row, 0, unroll=False)
            pltpu.sync_copy(yv, gt.at[pl.ds(g0 + k * EC, EC)])
            return 0

        lax.fori_loop(0, nchunks(h), chunk, 0, unroll=False)

    def run(xoff, y1t, y2t, gt, dvt, b1r):
        pltpu.sync_copy(b1r, b1v)
        for h in (0, 1):
            deg_scatter(h)
            deg_elementwise(h, xoff, y1t, dvt)
        for h in (0, 1):
            agg_scatter(h, y1t)
            layer1_elementwise(h, y1t, y2t, dvt)
        for h in (0, 1):
            agg_scatter(h, y2t)
            g_elementwise(h, y2t, gt, dvt)

    @pl.when(c == 0)
    def _():
        run(0, y1lo, y2lo, glo, dvlo, b1rlo)

    @pl.when(c == 1)
    def _():
        run(DH, y1hi, y2hi, ghi, dvhi, b1rhi)


BN = 1000


def _mm_body(x_ref, w_ref, o_ref):
    o_ref[...] = jnp.dot(x_ref[...], w_ref[...],
                         preferred_element_type=_f32)


def _tc_matmul(x, w):
    m, k = x.shape
    n = w.shape[1]
    return pl.pallas_call(
        _mm_body,
        grid=(m // BN,),
        in_specs=[pl.BlockSpec((BN, k), lambda i: (i, 0)),
                  pl.BlockSpec((k, n), lambda i: (0, 0))],
        out_specs=pl.BlockSpec((BN, n), lambda i: (i, 0)),
        out_shape=jax.ShapeDtypeStruct((m, n), _f32),
    )(x, w)


def _mask_body(src_ref, dst_ref, s0_ref, s1_ref, d0_ref, d1_ref):
    src = src_ref[...]
    dst = dst_ref[...]
    neg1 = jnp.full(src.shape, -1, jnp.int32)
    v0 = dst < NH
    s0_ref[...] = jnp.where(v0, src, neg1)
    d0_ref[...] = jnp.where(v0, dst, neg1)
    s1_ref[...] = jnp.where(v0, neg1, src)
    d1_ref[...] = jnp.where(v0, neg1, dst - NH)


def _tc_mask(src2, dst2):
    nrows = src2.shape[0]
    bspec = pl.BlockSpec((nrows // 10, CK), lambda i: (i, 0))
    return pl.pallas_call(
        _mask_body,
        grid=(10,),
        in_specs=[bspec, bspec],
        out_specs=[bspec] * 4,
        out_shape=[jax.ShapeDtypeStruct((nrows, CK), jnp.int32)] * 4,
    )(src2, dst2)


def _final_body(glo_ref, ghi_ref, wmu_a_ref, wmu_b_ref, bmu_ref,
                wls_a_ref, wls_b_ref, bls_ref, mu_ref, ls_ref):
    glo = glo_ref[...]
    ghi = ghi_ref[...]
    mu_ref[...] = (jnp.dot(glo, wmu_a_ref[...], preferred_element_type=_f32)
                   + jnp.dot(ghi, wmu_b_ref[...], preferred_element_type=_f32)
                   + bmu_ref[...])
    ls_ref[...] = (jnp.dot(glo, wls_a_ref[...], preferred_element_type=_f32)
                   + jnp.dot(ghi, wls_b_ref[...], preferred_element_type=_f32)
                   + bls_ref[...])


def _tc_final(glo, ghi, wmu_a, wmu_b, bmu, wls_a, wls_b, bls):
    hspec = pl.BlockSpec((BN, DH), lambda i: (i, 0))
    wspec = pl.BlockSpec((DH, D_OUT), lambda i: (0, 0))
    bspec = pl.BlockSpec((1, D_OUT), lambda i: (0, 0))
    return pl.pallas_call(
        _final_body,
        grid=(N // BN,),
        in_specs=[hspec, hspec, wspec, wspec, bspec, wspec, wspec, bspec],
        out_specs=[pl.BlockSpec((BN, D_OUT), lambda i: (i, 0))] * 2,
        out_shape=[jax.ShapeDtypeStruct((N, D_OUT), _f32)] * 2,
    )(glo, ghi, wmu_a, wmu_b, bmu, wls_a, wls_b, bls)


def kernel(x, edge_index, W1, b1, Wmu, bmu, Wls, bls):
    src2 = edge_index[0].reshape(E // CK, CK)
    dst2 = edge_index[1].reshape(E // CK, CK)

    sm0, sm1, dm0, dm1 = _tc_mask(src2, dst2)
    idx4 = lambda a: a.reshape(NS, AGG_CHUNKS, 1, CK)

    xw = _tc_matmul(x, W1)

    b1rlo = jnp.broadcast_to(b1[:DH], (16, DH))
    b1rhi = jnp.broadcast_to(b1[DH:], (16, DH))
    outs = _mega_kernel(idx4(sm0), idx4(sm1), idx4(dm0), idx4(dm1), xw,
                        b1rlo, b1rhi)
    glo, ghi = outs[4], outs[5]

    mu, ls = _tc_final(glo, ghi,
                       Wmu[:DH], Wmu[DH:], bmu.reshape(1, D_OUT),
                       Wls[:DH], Wls[DH:], bls.reshape(1, D_OUT))
    return (mu, ls)
